# baseline (device time: 58664 ns/iter reference)
import jax
import jax.numpy as jnp
from jax import lax
from jax.experimental import pallas as pl
from jax.experimental.pallas import tpu as pltpu

N_CHUNKS = 8


def kernel(x, W):
    t, d = x.shape
    _, v_half = W.shape
    chunk = v_half // N_CHUNKS

    def body(x_ref, w_ref, out_ref, send_buf, recv_buf, stats_send,
             stats_recv, send_sems, recv_sems, stats_sems):
        my_x = lax.axis_index("x")
        my_y = lax.axis_index("y")
        my_z = lax.axis_index("z")
        peer_z = 1 - my_z

        barrier_sem = pltpu.get_barrier_semaphore()
        pl.semaphore_signal(
            barrier_sem, inc=1,
            device_id=(my_x, my_y, peer_z),
            device_id_type=pl.DeviceIdType.MESH,
        )
        pl.semaphore_wait(barrier_sem, 1)

        rdmas = [
            pltpu.make_async_remote_copy(
                src_ref=send_buf.at[c],
                dst_ref=recv_buf.at[c],
                send_sem=send_sems.at[c],
                recv_sem=recv_sems.at[c],
                device_id=(my_x, my_y, peer_z),
                device_id_type=pl.DeviceIdType.MESH,
            )
            for c in range(N_CHUNKS)
        ]
        stats_rdma = pltpu.make_async_remote_copy(
            src_ref=stats_send,
            dst_ref=stats_recv,
            send_sem=stats_sems.at[0],
            recv_sem=stats_sems.at[1],
            device_id=(my_x, my_y, peer_z),
            device_id_type=pl.DeviceIdType.MESH,
        )

        send_buf[0, :, :] = jnp.dot(
            x_ref[:, :], w_ref[:, 0:chunk],
            preferred_element_type=jnp.float32,
        )
        rdmas[0].start()

        for c in range(1, N_CHUNKS):
            cols = slice(c * chunk, (c + 1) * chunk)
            send_buf[c, :, :] = jnp.dot(
                x_ref[:, :], w_ref[:, cols],
                preferred_element_type=jnp.float32,
            )

        m_l = jnp.max(send_buf[0, :, :], axis=-1, keepdims=True)
        for c in range(1, N_CHUNKS):
            m_l = jnp.maximum(
                m_l, jnp.max(send_buf[c, :, :], axis=-1, keepdims=True))
        s_l = jnp.zeros((t, 1), jnp.float32)
        for c in range(N_CHUNKS):
            s_l = s_l + jnp.sum(
                jnp.exp(send_buf[c, :, :] - m_l), axis=-1, keepdims=True)
        stats_send[:, 0:1] = m_l
        stats_send[:, 1:2] = s_l

        stats_rdma.start()
        for c in range(1, N_CHUNKS):
            rdmas[c].start()

        stats_rdma.wait_recv()
        m_p = stats_recv[:, 0:1]
        s_p = stats_recv[:, 1:2]
        m = jnp.maximum(m_l, m_p)
        inv = 1.0 / (s_l * jnp.exp(m_l - m) + s_p * jnp.exp(m_p - m))

        for c in range(N_CHUNKS):
            out_ref[:, pl.ds(my_z * v_half + c * chunk, chunk)] = (
                jnp.exp(send_buf[c, :, :] - m) * inv)

        for c in range(N_CHUNKS):
            rdmas[c].wait_recv()
            out_ref[:, pl.ds(peer_z * v_half + c * chunk, chunk)] = (
                jnp.exp(recv_buf[c, :, :] - m) * inv)

        stats_rdma.wait_send()
        for rdma in rdmas:
            rdma.wait_send()

    return pl.pallas_call(
        body,
        out_shape=jax.ShapeDtypeStruct((t, 2 * v_half), jnp.float32),
        in_specs=[
            pl.BlockSpec(memory_space=pltpu.VMEM),
            pl.BlockSpec(memory_space=pltpu.VMEM),
        ],
        out_specs=pl.BlockSpec(memory_space=pltpu.VMEM),
        scratch_shapes=[
            pltpu.VMEM((N_CHUNKS, t, chunk), jnp.float32),
            pltpu.VMEM((N_CHUNKS, t, chunk), jnp.float32),
            pltpu.VMEM((t, 8), jnp.float32),
            pltpu.VMEM((t, 8), jnp.float32),
            pltpu.SemaphoreType.DMA((N_CHUNKS,)),
            pltpu.SemaphoreType.DMA((N_CHUNKS,)),
            pltpu.SemaphoreType.DMA((2,)),
        ],
        compiler_params=pltpu.CompilerParams(collective_id=0),
    )(x, W)


# device time: 46316 ns/iter; 1.2666x vs baseline; 1.2666x over previous
import jax
import jax.numpy as jnp
from jax import lax
from jax.experimental import pallas as pl
from jax.experimental.pallas import tpu as pltpu


def kernel(x, W):
    t, d = x.shape
    _, v_half = W.shape
    qcols = v_half // 4
    hcols = qcols // 2

    def body(x_ref, w_ref, out_ref, logits, zsend, zrecv, xrecv, yrecv,
             drecv, stats_send, stats_recv, ssems, rsems, stats_sems):
        my_x = lax.axis_index("x")
        my_y = lax.axis_index("y")
        my_z = lax.axis_index("z")
        peer_z = 1 - my_z
        q_me = 2 * my_x + my_y
        q_x = 2 * (1 - my_x) + my_y
        q_y = 2 * my_x + (1 - my_y)
        q_d = 2 * (1 - my_x) + (1 - my_y)

        barrier_sem = pltpu.get_barrier_semaphore()
        for dev in [(my_x, my_y, peer_z), (1 - my_x, my_y, my_z),
                    (my_x, 1 - my_y, my_z)]:
            pl.semaphore_signal(
                barrier_sem, inc=1,
                device_id=dev, device_id_type=pl.DeviceIdType.MESH,
            )
        pl.semaphore_wait(barrier_sem, 3)

        z_peer = (my_x, my_y, peer_z)
        x_nbr = (1 - my_x, my_y, my_z)
        y_nbr = (my_x, 1 - my_y, my_z)

        def copy(src, dst, sem_i, dev):
            return pltpu.make_async_remote_copy(
                src_ref=src, dst_ref=dst,
                send_sem=ssems.at[sem_i], recv_sem=rsems.at[sem_i],
                device_id=dev, device_id_type=pl.DeviceIdType.MESH,
            )

        z_rdma = copy(zsend, zrecv, 0, z_peer)
        x1_rdma = copy(zrecv, xrecv, 1, x_nbr)
        y1_rdma = copy(zrecv, yrecv, 2, y_nbr)
        x2_rdma = copy(yrecv.at[:, 0:hcols], drecv.at[:, 0:hcols], 3, x_nbr)
        y2_rdma = copy(xrecv.at[:, hcols:qcols], drecv.at[:, hcols:qcols],
                       4, y_nbr)
        stats_rdma = pltpu.make_async_remote_copy(
            src_ref=stats_send, dst_ref=stats_recv,
            send_sem=stats_sems.at[0], recv_sem=stats_sems.at[1],
            device_id=z_peer, device_id_type=pl.DeviceIdType.MESH,
        )

        logits[:, :] = jnp.dot(x_ref[:, :], w_ref[:, :],
                               preferred_element_type=jnp.float32)
        zsend[:, :] = logits[:, pl.ds(q_me * qcols, qcols)]
        z_rdma.start()

        m_l = jnp.max(logits[:, :], axis=-1, keepdims=True)
        s_l = jnp.sum(jnp.exp(logits[:, :] - m_l), axis=-1, keepdims=True)
        stats_send[:, 0:1] = m_l
        stats_send[:, 1:2] = s_l
        stats_rdma.start()

        z_rdma.wait_recv()
        x1_rdma.start()
        y1_rdma.start()

        stats_rdma.wait_recv()
        m_p = stats_recv[:, 0:1]
        s_p = stats_recv[:, 1:2]
        m = jnp.maximum(m_l, m_p)
        inv = 1.0 / (s_l * jnp.exp(m_l - m) + s_p * jnp.exp(m_p - m))

        out_ref[:, pl.ds(my_z * v_half, v_half)] = (
            jnp.exp(logits[:, :] - m) * inv)
        peer_base = peer_z * v_half
        out_ref[:, pl.ds(peer_base + q_me * qcols, qcols)] = (
            jnp.exp(zrecv[:, :] - m) * inv)

        x1_rdma.wait_recv()
        y1_rdma.wait_recv()
        x2_rdma.start()
        y2_rdma.start()

        out_ref[:, pl.ds(peer_base + q_x * qcols, qcols)] = (
            jnp.exp(xrecv[:, :] - m) * inv)
        out_ref[:, pl.ds(peer_base + q_y * qcols, qcols)] = (
            jnp.exp(yrecv[:, :] - m) * inv)

        x2_rdma.wait_recv()
        y2_rdma.wait_recv()
        out_ref[:, pl.ds(peer_base + q_d * qcols, qcols)] = (
            jnp.exp(drecv[:, :] - m) * inv)

        for rdma in [z_rdma, x1_rdma, y1_rdma, x2_rdma, y2_rdma,
                     stats_rdma]:
            rdma.wait_send()

    return pl.pallas_call(
        body,
        out_shape=jax.ShapeDtypeStruct((t, 2 * v_half), jnp.float32),
        in_specs=[
            pl.BlockSpec(memory_space=pltpu.VMEM),
            pl.BlockSpec(memory_space=pltpu.VMEM),
        ],
        out_specs=pl.BlockSpec(memory_space=pltpu.VMEM),
        scratch_shapes=[
            pltpu.VMEM((t, v_half), jnp.float32),
            pltpu.VMEM((t, qcols), jnp.float32),
            pltpu.VMEM((t, qcols), jnp.float32),
            pltpu.VMEM((t, qcols), jnp.float32),
            pltpu.VMEM((t, qcols), jnp.float32),
            pltpu.VMEM((t, qcols), jnp.float32),
            pltpu.VMEM((t, 8), jnp.float32),
            pltpu.VMEM((t, 8), jnp.float32),
            pltpu.SemaphoreType.DMA((5,)),
            pltpu.SemaphoreType.DMA((5,)),
            pltpu.SemaphoreType.DMA((2,)),
        ],
        compiler_params=pltpu.CompilerParams(collective_id=0),
    )(x, W)


# device time: 39479 ns/iter; 1.4860x vs baseline; 1.1732x over previous
import jax
import jax.numpy as jnp
from jax import lax
from jax.experimental import pallas as pl
from jax.experimental.pallas import tpu as pltpu

SUB = 4


def kernel(x, W):
    t, d = x.shape
    _, v_half = W.shape
    qcols = v_half // 4
    hcols = qcols // 2
    w = qcols // SUB

    def body(x_ref, w_ref, out_ref, logits, zsend, zrecv, xrecv, yrecv,
             drecv, stats_send, stats_recv, zs_sems, zr_sems, x1s_sems,
             x1r_sems, y1s_sems, y1r_sems, p2_sems, stats_sems):
        my_x = lax.axis_index("x")
        my_y = lax.axis_index("y")
        my_z = lax.axis_index("z")
        peer_z = 1 - my_z
        q_me = 2 * my_x + my_y
        q_x = 2 * (1 - my_x) + my_y
        q_y = 2 * my_x + (1 - my_y)
        q_d = 2 * (1 - my_x) + (1 - my_y)

        z_peer = (my_x, my_y, peer_z)
        x_nbr = (1 - my_x, my_y, my_z)
        y_nbr = (my_x, 1 - my_y, my_z)

        barrier_sem = pltpu.get_barrier_semaphore()
        for dev in [z_peer, x_nbr, y_nbr]:
            pl.semaphore_signal(
                barrier_sem, inc=1,
                device_id=dev, device_id_type=pl.DeviceIdType.MESH,
            )
        pl.semaphore_wait(barrier_sem, 3)

        def sub(buf, s):
            return buf.at[:, s * w:(s + 1) * w]

        def copy(src, dst, send_sem, recv_sem, dev):
            return pltpu.make_async_remote_copy(
                src_ref=src, dst_ref=dst,
                send_sem=send_sem, recv_sem=recv_sem,
                device_id=dev, device_id_type=pl.DeviceIdType.MESH,
            )

        z_rdmas = [
            copy(sub(zsend, s), sub(zrecv, s), zs_sems.at[s], zr_sems.at[s],
                 z_peer)
            for s in range(SUB)
        ]
        x1_rdmas = [
            copy(sub(zrecv, s), sub(xrecv, s), x1s_sems.at[s],
                 x1r_sems.at[s], x_nbr)
            for s in range(SUB)
        ]
        y1_rdmas = [
            copy(sub(zrecv, s), sub(yrecv, s), y1s_sems.at[s],
                 y1r_sems.at[s], y_nbr)
            for s in range(SUB)
        ]
        x2_rdma = copy(yrecv.at[:, 0:hcols], drecv.at[:, 0:hcols],
                       p2_sems.at[0], p2_sems.at[1], x_nbr)
        y2_rdma = copy(xrecv.at[:, hcols:qcols], drecv.at[:, hcols:qcols],
                       p2_sems.at[2], p2_sems.at[3], y_nbr)
        stats_rdma = pltpu.make_async_remote_copy(
            src_ref=stats_send, dst_ref=stats_recv,
            send_sem=stats_sems.at[0], recv_sem=stats_sems.at[1],
            device_id=z_peer, device_id_type=pl.DeviceIdType.MESH,
        )

        logits[:, :] = jnp.dot(x_ref[:, :], w_ref[:, :],
                               preferred_element_type=jnp.float32)
        zsend[:, :] = logits[:, pl.ds(q_me * qcols, qcols)]
        z_rdmas[0].start()
        m_l = jnp.max(logits[:, :], axis=-1, keepdims=True)
        s_l = jnp.sum(jnp.exp(logits[:, :] - m_l), axis=-1, keepdims=True)
        stats_send[:, 0:1] = m_l
        stats_send[:, 1:2] = s_l
        stats_rdma.start()
        for s in range(1, SUB):
            z_rdmas[s].start()

        for s in range(SUB):
            z_rdmas[s].wait_recv()
            x1_rdmas[s].start()
            y1_rdmas[s].start()
            if s == 0:
                stats_rdma.wait_recv()
                m_p = stats_recv[:, 0:1]
                s_p = stats_recv[:, 1:2]
                m = jnp.maximum(m_l, m_p)
                inv = 1.0 / (s_l * jnp.exp(m_l - m)
                             + s_p * jnp.exp(m_p - m))
                out_ref[:, pl.ds(my_z * v_half, v_half)] = (
                    jnp.exp(logits[:, :] - m) * inv)

        peer_base = peer_z * v_half
        out_ref[:, pl.ds(peer_base + q_me * qcols, qcols)] = (
            jnp.exp(zrecv[:, :] - m) * inv)

        for s in range(SUB // 2):
            y1_rdmas[s].wait_recv()
        x2_rdma.start()
        for s in range(SUB // 2, SUB):
            x1_rdmas[s].wait_recv()
        y2_rdma.start()

        for s in range(SUB // 2):
            x1_rdmas[s].wait_recv()
        out_ref[:, pl.ds(peer_base + q_x * qcols, qcols)] = (
            jnp.exp(xrecv[:, :] - m) * inv)
        for s in range(SUB // 2, SUB):
            y1_rdmas[s].wait_recv()
        out_ref[:, pl.ds(peer_base + q_y * qcols, qcols)] = (
            jnp.exp(yrecv[:, :] - m) * inv)

        x2_rdma.wait_recv()
        y2_rdma.wait_recv()
        out_ref[:, pl.ds(peer_base + q_d * qcols, qcols)] = (
            jnp.exp(drecv[:, :] - m) * inv)

        for rdma in z_rdmas + x1_rdmas + y1_rdmas:
            rdma.wait_send()
        x2_rdma.wait_send()
        y2_rdma.wait_send()
        stats_rdma.wait_send()

    return pl.pallas_call(
        body,
        out_shape=jax.ShapeDtypeStruct((t, 2 * v_half), jnp.float32),
        in_specs=[
            pl.BlockSpec(memory_space=pltpu.VMEM),
            pl.BlockSpec(memory_space=pltpu.VMEM),
        ],
        out_specs=pl.BlockSpec(memory_space=pltpu.VMEM),
        scratch_shapes=[
            pltpu.VMEM((t, v_half), jnp.float32),
            pltpu.VMEM((t, qcols), jnp.float32),
            pltpu.VMEM((t, qcols), jnp.float32),
            pltpu.VMEM((t, qcols), jnp.float32),
            pltpu.VMEM((t, qcols), jnp.float32),
            pltpu.VMEM((t, qcols), jnp.float32),
            pltpu.VMEM((t, 8), jnp.float32),
            pltpu.VMEM((t, 8), jnp.float32),
            pltpu.SemaphoreType.DMA((SUB,)),
            pltpu.SemaphoreType.DMA((SUB,)),
            pltpu.SemaphoreType.DMA((SUB,)),
            pltpu.SemaphoreType.DMA((SUB,)),
            pltpu.SemaphoreType.DMA((SUB,)),
            pltpu.SemaphoreType.DMA((SUB,)),
            pltpu.SemaphoreType.DMA((4,)),
            pltpu.SemaphoreType.DMA((2,)),
        ],
        compiler_params=pltpu.CompilerParams(collective_id=0),
    )(x, W)


# device time: 38736 ns/iter; 1.5145x vs baseline; 1.0192x over previous
import jax
import jax.numpy as jnp
from jax import lax
from jax.experimental import pallas as pl
from jax.experimental.pallas import tpu as pltpu

SUB = 8


def kernel(x, W):
    t, d = x.shape
    _, v_half = W.shape
    qcols = v_half // 4
    hcols = qcols // 2
    w = qcols // SUB

    def body(x_ref, w_ref, out_ref, logits, zsend, zrecv, xrecv, yrecv,
             drecv, stats_send, stats_recv, zs_sems, zr_sems, x1s_sems,
             x1r_sems, y1s_sems, y1r_sems, p2_sems, stats_sems):
        my_x = lax.axis_index("x")
        my_y = lax.axis_index("y")
        my_z = lax.axis_index("z")
        peer_z = 1 - my_z
        q_me = 2 * my_x + my_y
        q_x = 2 * (1 - my_x) + my_y
        q_y = 2 * my_x + (1 - my_y)
        q_d = 2 * (1 - my_x) + (1 - my_y)

        z_peer = (my_x, my_y, peer_z)
        x_nbr = (1 - my_x, my_y, my_z)
        y_nbr = (my_x, 1 - my_y, my_z)

        barrier_sem = pltpu.get_barrier_semaphore()
        for dev in [z_peer, x_nbr, y_nbr]:
            pl.semaphore_signal(
                barrier_sem, inc=1,
                device_id=dev, device_id_type=pl.DeviceIdType.MESH,
            )

        def sub(buf, s):
            return buf.at[:, s * w:(s + 1) * w]

        def copy(src, dst, send_sem, recv_sem, dev):
            return pltpu.make_async_remote_copy(
                src_ref=src, dst_ref=dst,
                send_sem=send_sem, recv_sem=recv_sem,
                device_id=dev, device_id_type=pl.DeviceIdType.MESH,
            )

        z_rdmas = [
            copy(sub(zsend, s), sub(zrecv, s), zs_sems.at[s], zr_sems.at[s],
                 z_peer)
            for s in range(SUB)
        ]
        x1_rdmas = [
            copy(sub(zrecv, s), sub(xrecv, s), x1s_sems.at[s],
                 x1r_sems.at[s], x_nbr)
            for s in range(SUB)
        ]
        y1_rdmas = [
            copy(sub(zrecv, s), sub(yrecv, s), y1s_sems.at[s],
                 y1r_sems.at[s], y_nbr)
            for s in range(SUB)
        ]
        x2_rdma = copy(yrecv.at[:, 0:hcols], drecv.at[:, 0:hcols],
                       p2_sems.at[0], p2_sems.at[1], x_nbr)
        y2_rdma = copy(xrecv.at[:, hcols:qcols], drecv.at[:, hcols:qcols],
                       p2_sems.at[2], p2_sems.at[3], y_nbr)
        stats_rdma = pltpu.make_async_remote_copy(
            src_ref=stats_send, dst_ref=stats_recv,
            send_sem=stats_sems.at[0], recv_sem=stats_sems.at[1],
            device_id=z_peer, device_id_type=pl.DeviceIdType.MESH,
        )

        logits[:, :] = jnp.dot(x_ref[:, :], w_ref[:, :],
                               preferred_element_type=jnp.float32)
        zsend[:, :] = logits[:, pl.ds(q_me * qcols, qcols)]
        m_l = jnp.max(logits[:, :], axis=-1, keepdims=True)
        s_l = jnp.sum(jnp.exp(logits[:, :] - m_l), axis=-1, keepdims=True)
        stats_send[:, 0:1] = m_l
        stats_send[:, 1:2] = s_l
        pl.semaphore_wait(barrier_sem, 3)
        z_rdmas[0].start()
        z_rdmas[1].start()
        stats_rdma.start()
        for s in range(2, SUB):
            z_rdmas[s].start()

        for s in range(SUB):
            z_rdmas[s].wait_recv()
            x1_rdmas[s].start()
            y1_rdmas[s].start()

        for s in range(SUB // 2):
            y1_rdmas[s].wait_recv()
        x2_rdma.start()

        stats_rdma.wait_recv()
        m_p = stats_recv[:, 0:1]
        s_p = stats_recv[:, 1:2]
        m = jnp.maximum(m_l, m_p)
        inv = 1.0 / (s_l * jnp.exp(m_l - m) + s_p * jnp.exp(m_p - m))
        out_ref[:, pl.ds(my_z * v_half, v_half)] = (
            jnp.exp(logits[:, :] - m) * inv)
        peer_base = peer_z * v_half
        out_ref[:, pl.ds(peer_base + q_me * qcols, qcols)] = (
            jnp.exp(zrecv[:, :] - m) * inv)

        for s in range(SUB // 2, SUB):
            x1_rdmas[s].wait_recv()
        y2_rdma.start()

        for s in range(SUB // 2):
            x1_rdmas[s].wait_recv()
        out_ref[:, pl.ds(peer_base + q_x * qcols, qcols)] = (
            jnp.exp(xrecv[:, :] - m) * inv)
        for s in range(SUB // 2, SUB):
            y1_rdmas[s].wait_recv()
        out_ref[:, pl.ds(peer_base + q_y * qcols, qcols)] = (
            jnp.exp(yrecv[:, :] - m) * inv)

        x2_rdma.wait_recv()
        y2_rdma.wait_recv()
        out_ref[:, pl.ds(peer_base + q_d * qcols, qcols)] = (
            jnp.exp(drecv[:, :] - m) * inv)

        for rdma in z_rdmas + x1_rdmas + y1_rdmas:
            rdma.wait_send()
        x2_rdma.wait_send()
        y2_rdma.wait_send()
        stats_rdma.wait_send()

    return pl.pallas_call(
        body,
        out_shape=jax.ShapeDtypeStruct((t, 2 * v_half), jnp.float32),
        in_specs=[
            pl.BlockSpec(memory_space=pltpu.VMEM),
            pl.BlockSpec(memory_space=pltpu.VMEM),
        ],
        out_specs=pl.BlockSpec(memory_space=pltpu.VMEM),
        scratch_shapes=[
            pltpu.VMEM((t, v_half), jnp.float32),
            pltpu.VMEM((t, qcols), jnp.float32),
            pltpu.VMEM((t, qcols), jnp.float32),
            pltpu.VMEM((t, qcols), jnp.float32),
            pltpu.VMEM((t, qcols), jnp.float32),
            pltpu.VMEM((t, qcols), jnp.float32),
            pltpu.VMEM((t, 8), jnp.float32),
            pltpu.VMEM((t, 8), jnp.float32),
            pltpu.SemaphoreType.DMA((SUB,)),
            pltpu.SemaphoreType.DMA((SUB,)),
            pltpu.SemaphoreType.DMA((SUB,)),
            pltpu.SemaphoreType.DMA((SUB,)),
            pltpu.SemaphoreType.DMA((SUB,)),
            pltpu.SemaphoreType.DMA((SUB,)),
            pltpu.SemaphoreType.DMA((4,)),
            pltpu.SemaphoreType.DMA((2,)),
        ],
        compiler_params=pltpu.CompilerParams(collective_id=0),
    )(x, W)
